# baseline (speedup 1.0000x reference)
import functools

import jax
import jax.numpy as jnp
from jax import lax
from jax.experimental import pallas as pl
from jax.experimental.pallas import tpu as pltpu
from jax.experimental.pallas import tpu_sc as plsc

F32 = jnp.float32
HI = lax.Precision.HIGHEST
DEF = lax.Precision.DEFAULT

N_NODES = 10240
N_EDGES = 163840
DIM = 128
BATCH = 8
NG0 = N_NODES // BATCH
N_OUT = 10


_NSC = 2
_NTILE = 16
_CH = 128
_ROWS_PER_TILE = N_NODES // _NTILE
_EDGES_PER_SC = N_EDGES // _NSC
_EDGES_PER_TILE = _EDGES_PER_SC // _NTILE
_NCHUNK = _EDGES_PER_TILE // _CH

_NBUF = 2
_NGRP = _NCHUNK // _NBUF


@functools.lru_cache(maxsize=1)
def _get_segsum_sc():
    mesh = plsc.VectorSubcoreMesh(
        core_axis_name="c", subcore_axis_name="s",
        num_cores=_NSC, num_subcores=_NTILE)

    @functools.partial(
        pl.kernel,
        out_type=jax.ShapeDtypeStruct((_NSC * N_NODES, DIM), F32),
        mesh=mesh,
        scratch_types=[
            pltpu.VMEM((_NCHUNK, _CH), jnp.int32),
            pltpu.VMEM((_NCHUNK, _CH), jnp.int32),
            pltpu.VMEM((_CH, DIM), F32),
            pltpu.VMEM((_CH, DIM), F32),
            pltpu.SemaphoreType.DMA,
            pltpu.SemaphoreType.DMA,
            pltpu.SemaphoreType.DMA,
            pltpu.SemaphoreType.DMA,
            pltpu.VMEM_SHARED((N_NODES, DIM), F32),
        ],
    )
    def _segsum_sc(x_hbm, src_hbm, dst_hbm, zero_hbm, out_hbm,
                   src_v, dst_v, rows0, rows1,
                   gsem0, gsem1, ssem0, ssem1, acc):
        rows = (rows0, rows1)
        gsem = (gsem0, gsem1)
        ssem = (ssem0, ssem1)
        c = lax.axis_index("c")
        s = lax.axis_index("s")
        r0 = s * _ROWS_PER_TILE
        crow = c * _NCHUNK * _NTILE + s * _NCHUNK
        pltpu.sync_copy(src_hbm.at[pl.ds(crow, _NCHUNK)], src_v)
        pltpu.sync_copy(dst_hbm.at[pl.ds(crow, _NCHUNK)], dst_v)
        pltpu.sync_copy(zero_hbm.at[pl.ds(r0, _ROWS_PER_TILE)],
                        acc.at[pl.ds(r0, _ROWS_PER_TILE)])
        plsc.subcore_barrier()

        def start_g(j, b):
            pltpu.async_copy(x_hbm.at[src_v.at[j]], rows[b], gsem[b])

        def wait_g(b):
            pltpu.make_async_copy(x_hbm.at[src_v.at[0]], rows[b],
                                  gsem[b]).wait()

        def start_s(j, b):
            pltpu.async_copy(rows[b], acc.at[dst_v.at[j]], ssem[b], add=True)

        def wait_s(b):
            pltpu.make_async_copy(rows[b], acc.at[dst_v.at[0]],
                                  ssem[b]).wait()

        start_g(0, 0)

        def group(g, carry):
            j0 = g * 2

            @pl.when(g > 0)
            def _():
                wait_s(1)

            start_g(j0 + 1, 1)
            wait_g(0)
            start_s(j0, 0)
            @pl.when(g < _NGRP - 1)
            def _():
                wait_s(0)
                start_g(j0 + 2, 0)

            wait_g(1)
            start_s(j0 + 1, 1)
            return carry

        lax.fori_loop(0, _NGRP, group, 0)
        wait_s(0)
        wait_s(1)
        plsc.subcore_barrier()
        pltpu.sync_copy(acc.at[pl.ds(r0, _ROWS_PER_TILE)],
                        out_hbm.at[pl.ds(c * N_NODES + r0, _ROWS_PER_TILE)])

    return _segsum_sc


def _eye(n):
    r = lax.broadcasted_iota(jnp.int32, (n, n), 0)
    c = lax.broadcasted_iota(jnp.int32, (n, n), 1)
    return jnp.where(r == c, jnp.float32(1.0), jnp.float32(0.0))


def _dotT(a, w):
    return lax.dot_general(a, w, (((1,), (1,)), ((), ())), precision=DEF)


def _dot(a, b):
    return lax.dot_general(a, b, (((1,), (0,)), ((), ())), precision=HI)


def _xdot(sel, v):
    bf = jnp.bfloat16
    s16 = sel.astype(bf)
    v1 = v.astype(bf)
    r1 = v - v1.astype(F32)
    v2 = r1.astype(bf)
    v3 = (r1 - v2.astype(F32)).astype(bf)

    def bdot(a, b):
        return lax.dot_general(a, b, (((1,), (0,)), ((), ())),
                               preferred_element_type=F32)

    return (bdot(s16, v1) + bdot(s16, v2)) + bdot(s16, v3)


def _pool_matrix(s_col, n, k):
    s_row = s_col.reshape(1, n)
    cmp = jnp.where(
        (s_col > s_row)
        | ((s_col == s_row)
           & (lax.broadcasted_iota(jnp.int32, (n, n), 0)
              < lax.broadcasted_iota(jnp.int32, (n, n), 1))),
        jnp.float32(1.0), jnp.float32(0.0))
    ones_row = jnp.full((1, n), 1.0, jnp.bfloat16)
    rank_row = lax.dot_general(ones_row, cmp.astype(jnp.bfloat16),
                               (((1,), (0,)), ((), ())),
                               preferred_element_type=F32)
    rit = lax.broadcasted_iota(jnp.int32, (k, n), 0).astype(F32)
    return jnp.where(rit == rank_row, jnp.float32(1.0), jnp.float32(0.0))


def _knn_adj(pp, k, kk, eye_k):
    d2 = None
    for d in range(3):
        cc = pp[:, d:d + 1]
        rr = cc.reshape(1, k)
        df = cc - rr
        sq = df * df
        d2 = sq if d2 is None else d2 + sq
    d2 = d2 + eye_k * jnp.float32(1e10)
    colI = lax.broadcasted_iota(jnp.int32, (k, k), 1).astype(F32)
    big = jnp.float32(3e38)
    D = d2
    for _ in range(kk):
        m = jnp.min(D, axis=1, keepdims=True)
        am = jnp.min(jnp.where(D == m, colI, jnp.float32(k)),
                     axis=1, keepdims=True)
        D = jnp.where(colI == am, big, D)
    return jnp.where(D == big, jnp.float32(1.0), jnp.float32(0.0))


def _tc_body(x_ref, pos_ref, pa_ref, pb_ref,
             wr0_ref, wo0_ref, br0_ref, p0_ref,
             wr1_ref, wo1_ref, br1_ref, p1_ref,
             wr2_ref, wo2_ref, br2_ref, p2_ref,
             w1_ref, b1_ref, w2_ref, b2_ref, w3_ref, b3_ref,
             out_ref):
    x = x_ref[...]
    agg = pa_ref[...] + pb_ref[...]
    wrs = (wr0_ref[...], wr1_ref[...], wr2_ref[...])
    wos = (wo0_ref[...], wo1_ref[...], wo2_ref[...])
    brs = (br0_ref[...], br1_ref[...], br2_ref[...])
    pvs = (p0_ref[...], p1_ref[...], p2_ref[...])

    h = jnp.maximum(_dotT(agg, wrs[0]) + brs[0] + _dotT(x, wos[0]),
                    jnp.float32(0.0))
    pos = pos_ref[...]

    plan = ((1280, 640, 6), (640, 320, 8), (320, 160, 0))
    reads = []
    for i, (n, k, kk) in enumerate(plan):
        pv = pvs[i]
        nrm = jnp.sqrt(jnp.sum(pv * pv))
        z = lax.dot_general(h, pv, (((1,), (0,)), ((), ())), precision=DEF)
        s = jnp.tanh(z / nrm)
        P = _pool_matrix(s, n, k)
        gath = _xdot(P, jnp.concatenate([h * s, pos], axis=1))
        xp = gath[:, :DIM]
        pp = gath[:, DIM:DIM + 3]
        mx = jnp.max(xp, axis=0, keepdims=True)
        mn = jnp.sum(xp, axis=0, keepdims=True) / jnp.float32(k)
        reads.append(jnp.concatenate([mx, mn], axis=1))
        if kk:
            A = _knn_adj(pp, k, kk, _eye(k))
            aggn = _xdot(A, xp)
            h = jnp.maximum(_dotT(aggn, wrs[i + 1]) + brs[i + 1]
                            + _dotT(xp, wos[i + 1]), jnp.float32(0.0))
            pos = pp

    hs = (reads[0] + reads[1]) + reads[2]
    t = jnp.maximum(_dotT(hs, w1_ref[...]) + b1_ref[...], jnp.float32(0.0))
    t = jnp.maximum(_dotT(t, w2_ref[...]) + b2_ref[...], jnp.float32(0.0))
    o = _dotT(t, w3_ref[...]) + b3_ref[...]
    out_ref[...] = o.reshape(1, 1, N_OUT)


def _tc_in_specs():
    def im_g(g):
        return (g, 0)

    def im_0(g):
        return (0, 0)

    specs = [
        pl.BlockSpec((NG0, DIM), im_g),
        pl.BlockSpec((NG0, 3), im_g),
        pl.BlockSpec((NG0, DIM), im_g),
        pl.BlockSpec((NG0, DIM), im_g),
    ]
    wshapes = [
        (DIM, DIM), (DIM, DIM), (1, DIM), (DIM, 1),
        (DIM, DIM), (DIM, DIM), (1, DIM), (DIM, 1),
        (DIM, DIM), (DIM, DIM), (1, DIM), (DIM, 1),
        (DIM, 2 * DIM), (1, DIM),
        (DIM // 2, DIM), (1, DIM // 2),
        (N_OUT, DIM // 2), (1, N_OUT),
    ]
    specs += [pl.BlockSpec(sh, im_0) for sh in wshapes]
    return specs


_TC_OUT_SPEC = pl.BlockSpec((1, 1, N_OUT), lambda g: (g, 0, 0))
_TC_OUT_SHAPE = jax.ShapeDtypeStruct((BATCH, 1, N_OUT), F32)


def _tc_forward(*arrays):
    return pl.pallas_call(
        _tc_body,
        grid=(BATCH,),
        in_specs=_tc_in_specs(),
        out_specs=_TC_OUT_SPEC,
        out_shape=_TC_OUT_SHAPE,
    )(*arrays)


def kernel(x, pos, edge_index, edge_attr, batch,
           Wr0, br0, Wo0, Wr1, br1, Wo1, Wr2, br2, Wo2,
           p0, p1, p2, W1, b1, W2, b2, W3, b3):
    src = edge_index[0].reshape(N_EDGES // _CH, _CH)
    dst = edge_index[1].reshape(N_EDGES // _CH, _CH)
    zeros = jnp.zeros((N_NODES, DIM), F32)
    parts = _get_segsum_sc()(x, src, dst, zeros)
    pa = parts[:N_NODES]
    pb = parts[N_NODES:]
    out3 = _tc_forward(
        x, pos, pa, pb,
        Wr0, Wo0, br0.reshape(1, DIM), p0.reshape(DIM, 1),
        Wr1, Wo1, br1.reshape(1, DIM), p1.reshape(DIM, 1),
        Wr2, Wo2, br2.reshape(1, DIM), p2.reshape(DIM, 1),
        W1, b1.reshape(1, DIM), W2, b2.reshape(1, DIM // 2),
        W3, b3.reshape(1, N_OUT))
    return out3.reshape(BATCH, N_OUT)


# speedup vs baseline: 1.0402x; 1.0402x over previous
import functools

import jax
import jax.numpy as jnp
from jax import lax
from jax.experimental import pallas as pl
from jax.experimental.pallas import tpu as pltpu
from jax.experimental.pallas import tpu_sc as plsc

F32 = jnp.float32
HI = lax.Precision.HIGHEST
DEF = lax.Precision.DEFAULT

N_NODES = 10240
N_EDGES = 163840
DIM = 128
BATCH = 8
NG0 = N_NODES // BATCH
N_OUT = 10


_NSC = 2
_NTILE = 16
_CH = 128
_ROWS_PER_TILE = N_NODES // _NTILE
_EDGES_PER_SC = N_EDGES // _NSC
_EDGES_PER_TILE = _EDGES_PER_SC // _NTILE
_NCHUNK = _EDGES_PER_TILE // _CH

_NBUF = 2
_NGRP = _NCHUNK // _NBUF


@functools.lru_cache(maxsize=1)
def _get_segsum_sc():
    mesh = plsc.VectorSubcoreMesh(
        core_axis_name="c", subcore_axis_name="s",
        num_cores=_NSC, num_subcores=_NTILE)

    @functools.partial(
        pl.kernel,
        out_type=jax.ShapeDtypeStruct((_NSC * N_NODES, DIM), F32),
        mesh=mesh,
        scratch_types=[
            pltpu.VMEM((_NCHUNK, _CH), jnp.int32),
            pltpu.VMEM((_NCHUNK, _CH), jnp.int32),
            pltpu.VMEM((_CH, DIM), F32),
            pltpu.VMEM((_CH, DIM), F32),
            pltpu.SemaphoreType.DMA,
            pltpu.SemaphoreType.DMA,
            pltpu.SemaphoreType.DMA,
            pltpu.SemaphoreType.DMA,
            pltpu.VMEM_SHARED((N_NODES, DIM), F32),
        ],
    )
    def _segsum_sc(x_hbm, src_hbm, dst_hbm, zero_hbm, out_hbm,
                   src_v, dst_v, rows0, rows1,
                   gsem0, gsem1, ssem0, ssem1, acc):
        rows = (rows0, rows1)
        gsem = (gsem0, gsem1)
        ssem = (ssem0, ssem1)
        c = lax.axis_index("c")
        s = lax.axis_index("s")
        r0 = s * _ROWS_PER_TILE
        crow = c * _NCHUNK * _NTILE + s * _NCHUNK
        pltpu.sync_copy(src_hbm.at[pl.ds(crow, _NCHUNK)], src_v)
        pltpu.sync_copy(dst_hbm.at[pl.ds(crow, _NCHUNK)], dst_v)
        pltpu.sync_copy(zero_hbm.at[pl.ds(r0, _ROWS_PER_TILE)],
                        acc.at[pl.ds(r0, _ROWS_PER_TILE)])
        plsc.subcore_barrier()

        def start_g(j, b):
            pltpu.async_copy(x_hbm.at[src_v.at[j]], rows[b], gsem[b])

        def wait_g(b):
            pltpu.make_async_copy(x_hbm.at[src_v.at[0]], rows[b],
                                  gsem[b]).wait()

        def start_s(j, b):
            pltpu.async_copy(rows[b], acc.at[dst_v.at[j]], ssem[b], add=True)

        def wait_s(b):
            pltpu.make_async_copy(rows[b], acc.at[dst_v.at[0]],
                                  ssem[b]).wait()

        start_g(0, 0)

        def group(g, carry):
            j0 = g * 2

            @pl.when(g > 0)
            def _():
                wait_s(1)

            start_g(j0 + 1, 1)
            wait_g(0)
            start_s(j0, 0)
            @pl.when(g < _NGRP - 1)
            def _():
                wait_s(0)
                start_g(j0 + 2, 0)

            wait_g(1)
            start_s(j0 + 1, 1)
            return carry

        lax.fori_loop(0, _NGRP, group, 0)
        wait_s(0)
        wait_s(1)
        plsc.subcore_barrier()
        pltpu.sync_copy(acc.at[pl.ds(r0, _ROWS_PER_TILE)],
                        out_hbm.at[pl.ds(c * N_NODES + r0, _ROWS_PER_TILE)])

    return _segsum_sc


def _eye(n):
    r = lax.broadcasted_iota(jnp.int32, (n, n), 0)
    c = lax.broadcasted_iota(jnp.int32, (n, n), 1)
    return jnp.where(r == c, jnp.float32(1.0), jnp.float32(0.0))


def _dotT(a, w):
    return lax.dot_general(a, w, (((1,), (1,)), ((), ())), precision=DEF)


def _dot(a, b):
    return lax.dot_general(a, b, (((1,), (0,)), ((), ())), precision=HI)


def _xdot(sel, v):
    bf = jnp.bfloat16
    s16 = sel.astype(bf)
    v1 = v.astype(bf)
    r1 = v - v1.astype(F32)
    v2 = r1.astype(bf)
    v3 = (r1 - v2.astype(F32)).astype(bf)

    def bdot(a, b):
        return lax.dot_general(a, b, (((1,), (0,)), ((), ())),
                               preferred_element_type=F32)

    return (bdot(s16, v1) + bdot(s16, v2)) + bdot(s16, v3)


def _pool_matrix(s_col, n, k):
    s_row = s_col.reshape(1, n)
    cmp = jnp.where(
        (s_col > s_row)
        | ((s_col == s_row)
           & (lax.broadcasted_iota(jnp.int32, (n, n), 0)
              < lax.broadcasted_iota(jnp.int32, (n, n), 1))),
        jnp.float32(1.0), jnp.float32(0.0))
    ones_row = jnp.full((1, n), 1.0, jnp.bfloat16)
    rank_row = lax.dot_general(ones_row, cmp.astype(jnp.bfloat16),
                               (((1,), (0,)), ((), ())),
                               preferred_element_type=F32)
    rit = lax.broadcasted_iota(jnp.int32, (k, n), 0).astype(F32)
    return jnp.where(rit == rank_row, jnp.float32(1.0), jnp.float32(0.0))


def _knn_adj(pp, k, kk, eye_k):
    d2 = None
    for d in range(3):
        cc = pp[:, d:d + 1]
        rr = cc.reshape(1, k)
        df = cc - rr
        sq = df * df
        d2 = sq if d2 is None else d2 + sq
    d2 = d2 + eye_k * jnp.float32(1e10)
    colI = lax.broadcasted_iota(jnp.int32, (k, k), 1).astype(F32)
    big = jnp.float32(3e38)
    D = d2
    for _ in range(kk):
        m = jnp.min(D, axis=1, keepdims=True)
        am = jnp.min(jnp.where(D == m, colI, jnp.float32(k)),
                     axis=1, keepdims=True)
        D = jnp.where(colI == am, big, D)
    return jnp.where(D == big, jnp.float32(1.0), jnp.float32(0.0))


def _tc_body(x_ref, pos_ref, parts_ref,
             wr0_ref, wo0_ref, br0_ref, p0_ref,
             wr1_ref, wo1_ref, br1_ref, p1_ref,
             wr2_ref, wo2_ref, br2_ref, p2_ref,
             w1_ref, b1_ref, w2_ref, b2_ref, w3_ref, b3_ref,
             out_ref):
    x = x_ref[...]
    agg = parts_ref[0] + parts_ref[1]
    wrs = (wr0_ref[...], wr1_ref[...], wr2_ref[...])
    wos = (wo0_ref[...], wo1_ref[...], wo2_ref[...])
    brs = (br0_ref[...], br1_ref[...], br2_ref[...])
    pvs = (p0_ref[...], p1_ref[...], p2_ref[...])

    h = jnp.maximum(_dotT(agg, wrs[0]) + brs[0] + _dotT(x, wos[0]),
                    jnp.float32(0.0))
    pos = pos_ref[...]

    plan = ((1280, 640, 6), (640, 320, 8), (320, 160, 0))
    reads = []
    for i, (n, k, kk) in enumerate(plan):
        pv = pvs[i]
        nrm = jnp.sqrt(jnp.sum(pv * pv))
        z = lax.dot_general(h, pv, (((1,), (0,)), ((), ())), precision=DEF)
        s = jnp.tanh(z / nrm)
        P = _pool_matrix(s, n, k)
        gath = _xdot(P, jnp.concatenate([h * s, pos], axis=1))
        xp = gath[:, :DIM]
        pp = gath[:, DIM:DIM + 3]
        mx = jnp.max(xp, axis=0, keepdims=True)
        mn = jnp.sum(xp, axis=0, keepdims=True) / jnp.float32(k)
        reads.append(jnp.concatenate([mx, mn], axis=1))
        if kk:
            A = _knn_adj(pp, k, kk, _eye(k))
            aggn = _xdot(A, xp)
            h = jnp.maximum(_dotT(aggn, wrs[i + 1]) + brs[i + 1]
                            + _dotT(xp, wos[i + 1]), jnp.float32(0.0))
            pos = pp

    hs = (reads[0] + reads[1]) + reads[2]
    t = jnp.maximum(_dotT(hs, w1_ref[...]) + b1_ref[...], jnp.float32(0.0))
    t = jnp.maximum(_dotT(t, w2_ref[...]) + b2_ref[...], jnp.float32(0.0))
    o = _dotT(t, w3_ref[...]) + b3_ref[...]
    out_ref[...] = o.reshape(1, 1, N_OUT)


def _tc_in_specs():
    def im_g(g):
        return (g, 0)

    def im_0(g):
        return (0, 0)

    specs = [
        pl.BlockSpec((NG0, DIM), im_g),
        pl.BlockSpec((NG0, 3), im_g),
        pl.BlockSpec((_NSC, NG0, DIM), lambda g: (0, g, 0)),
    ]
    wshapes = [
        (DIM, DIM), (DIM, DIM), (1, DIM), (DIM, 1),
        (DIM, DIM), (DIM, DIM), (1, DIM), (DIM, 1),
        (DIM, DIM), (DIM, DIM), (1, DIM), (DIM, 1),
        (DIM, 2 * DIM), (1, DIM),
        (DIM // 2, DIM), (1, DIM // 2),
        (N_OUT, DIM // 2), (1, N_OUT),
    ]
    specs += [pl.BlockSpec(sh, im_0) for sh in wshapes]
    return specs


_TC_OUT_SPEC = pl.BlockSpec((1, 1, N_OUT), lambda g: (g, 0, 0))
_TC_OUT_SHAPE = jax.ShapeDtypeStruct((BATCH, 1, N_OUT), F32)


def _tc_forward(*arrays):
    return pl.pallas_call(
        _tc_body,
        grid=(BATCH,),
        in_specs=_tc_in_specs(),
        out_specs=_TC_OUT_SPEC,
        out_shape=_TC_OUT_SHAPE,
    )(*arrays)


def kernel(x, pos, edge_index, edge_attr, batch,
           Wr0, br0, Wo0, Wr1, br1, Wo1, Wr2, br2, Wo2,
           p0, p1, p2, W1, b1, W2, b2, W3, b3):
    src = edge_index[0].reshape(N_EDGES // _CH, _CH)
    dst = edge_index[1].reshape(N_EDGES // _CH, _CH)
    zeros = jnp.zeros((N_NODES, DIM), F32)
    parts = _get_segsum_sc()(x, src, dst, zeros)
    parts = parts.reshape(_NSC, N_NODES, DIM)
    out3 = _tc_forward(
        x, pos, parts,
        Wr0, Wo0, br0.reshape(1, DIM), p0.reshape(DIM, 1),
        Wr1, Wo1, br1.reshape(1, DIM), p1.reshape(DIM, 1),
        Wr2, Wo2, br2.reshape(1, DIM), p2.reshape(DIM, 1),
        W1, b1.reshape(1, DIM), W2, b2.reshape(1, DIM // 2),
        W3, b3.reshape(1, N_OUT))
    return out3.reshape(BATCH, N_OUT)
